# baseline (speedup 1.0000x reference)
import functools

import jax
import jax.numpy as jnp
from jax import lax
from jax.experimental import pallas as pl
from jax.experimental.pallas import tpu as pltpu
from jax.experimental.pallas import tpu_sc as plsc

N_FIELDS = 26
VOCAB = 100000
EMB_DIM = 32
BATCH = 16384
OUT_FEATURES = 128
IN_FEAT = N_FIELDS * EMB_DIM

_NW = 32
_ROWS = BATCH * N_FIELDS
_RPW = _ROWS // _NW
_G = 128
_NG = _RPW // _G
_NPAIR = _NG // 2


def _gather_body(tab_hbm, idx_hbm, out_hbm, idx_v, buf0, buf1, sem0, sem1):
    nc = lax.axis_size("c")
    wid = lax.axis_index("s") * nc + lax.axis_index("c")
    pltpu.sync_copy(idx_hbm.at[wid], idx_v)
    base = wid * _RPW

    pltpu.async_copy(tab_hbm.at[idx_v.at[0]], buf0, sem0)

    def body(i, carry):
        a = 2 * i
        pltpu.async_copy(tab_hbm.at[idx_v.at[a + 1]], buf1, sem1)
        pltpu.make_async_copy(tab_hbm.at[idx_v.at[a]], buf0, sem0).wait()
        pltpu.sync_copy(buf0, out_hbm.at[pl.ds(base + a * _G, _G)])

        @pl.when(i < _NPAIR - 1)
        def _():
            pltpu.async_copy(tab_hbm.at[idx_v.at[a + 2]], buf0, sem0)

        pltpu.make_async_copy(tab_hbm.at[idx_v.at[a + 1]], buf1, sem1).wait()
        pltpu.sync_copy(buf1, out_hbm.at[pl.ds(base + (a + 1) * _G, _G)])
        return carry

    lax.fori_loop(0, _NPAIR, body, 0)


@functools.partial(
    pl.kernel,
    out_type=jax.ShapeDtypeStruct((_ROWS, EMB_DIM), jnp.float32),
    mesh=plsc.VectorSubcoreMesh(core_axis_name="c", subcore_axis_name="s"),
    scratch_types=[
        pltpu.VMEM((_NG, _G), jnp.int32),
        pltpu.VMEM((_G, EMB_DIM), jnp.float32),
        pltpu.VMEM((_G, EMB_DIM), jnp.float32),
        pltpu.SemaphoreType.DMA,
        pltpu.SemaphoreType.DMA,
    ],
    compiler_params=pltpu.CompilerParams(use_tc_tiling_on_sc=False),
)
def _sc_gather(tab_hbm, idx_hbm, out_hbm, idx_v, buf0, buf1, sem0, sem1):
    _gather_body(tab_hbm, idx_hbm, out_hbm, idx_v, buf0, buf1, sem0, sem1)


_VQ = VOCAB // 4


_NT = VOCAB // 512


def _tr_body(in_ref, out_ref):
    ident = jnp.eye(128, dtype=jnp.float32)
    dn = (((0,), (0,)), ((), ()))

    def body(i, carry):
        for u in range(14):
            t = 14 * i + u
            base = 512 * t
            xs = jnp.concatenate(
                [in_ref[0, :, pl.ds(base + 128 * a, 128)] for a in range(4)],
                axis=0,
            )
            out_ref[pl.ds(128 * t, 128), :] = lax.dot_general(
                xs, ident, dn, preferred_element_type=jnp.float32
            )
        return carry

    lax.fori_loop(0, _NT // 14, body, 0)
    for t in range(14 * (_NT // 14), _NT):
        base = 512 * t
        xs = jnp.concatenate(
            [in_ref[0, :, base + 128 * a:base + 128 * (a + 1)]
             for a in range(4)],
            axis=0,
        )
        out_ref[128 * t:128 * (t + 1), :] = lax.dot_general(
            xs, ident, dn, preferred_element_type=jnp.float32
        )
    tb = 512 * _NT
    xt = jnp.concatenate(
        [in_ref[0, :, tb + 40 * a:tb + 40 * (a + 1)] for a in range(4)],
        axis=0,
    )
    out_ref[128 * _NT:_VQ, :] = lax.dot_general(
        xt, ident, dn, preferred_element_type=jnp.float32
    )


def _tc_transpose(tabT):
    rows_per_field = _VQ
    return pl.pallas_call(
        _tr_body,
        grid=(N_FIELDS,),
        in_specs=[pl.BlockSpec((1, EMB_DIM, VOCAB), lambda f: (f, 0, 0))],
        out_specs=pl.BlockSpec((rows_per_field, 128), lambda f: (f, 0)),
        out_shape=jax.ShapeDtypeStruct((N_FIELDS * rows_per_field, 128),
                                       jnp.float32),
    )(tabT)


_NP = BATCH // 2
_NJ = 2 * IN_FEAT // 128
_PB = 1024

_PERM52 = [
    (128 * j + 32 * a2) // IN_FEAT * N_FIELDS
    + ((128 * j + 32 * a2) % IN_FEAT) // EMB_DIM
    for j in range(_NJ) for a2 in range(4)
]


def _mm2_body(*refs):
    e_refs, w_ref, b_ref, o_ref = refs[:_NJ], refs[_NJ], refs[_NJ + 1], refs[-1]
    acc = b_ref[...].astype(jnp.float32)
    for j in range(_NJ):
        acc = acc + jnp.dot(
            e_refs[j][...], w_ref[128 * j:128 * (j + 1), :],
            preferred_element_type=jnp.float32,
        )
    o_ref[...] = acc


def _tc_matmul(emb2, W2, b2):
    in_specs = [
        pl.BlockSpec((_PB, 128), functools.partial(lambda j, pb: (j * (_NP // _PB) + pb, 0), j))
        for j in range(_NJ)
    ]
    in_specs.append(pl.BlockSpec((128 * _NJ, 2 * OUT_FEATURES), lambda pb: (0, 0)))
    in_specs.append(pl.BlockSpec((1, 2 * OUT_FEATURES), lambda pb: (0, 0)))
    out = pl.pallas_call(
        _mm2_body,
        grid=(_NP // _PB,),
        in_specs=in_specs,
        out_specs=pl.BlockSpec((_PB, 2 * OUT_FEATURES), lambda pb: (pb, 0)),
        out_shape=jax.ShapeDtypeStruct((_NP, 2 * OUT_FEATURES), jnp.float32),
    )(*([emb2] * _NJ), W2, b2)
    return out.reshape(BATCH, OUT_FEATURES)


def kernel(x, tables, W, b):
    tabT = jnp.transpose(tables, (0, 2, 1))
    tab_flat = _tc_transpose(tabT).reshape(N_FIELDS * VOCAB, EMB_DIM)
    xi = x.astype(jnp.int32)
    vt = xi - 512 * _NT
    r_main = 128 * (xi // 512) + xi % 128
    a_main = (xi // 128) % 4
    r_tail = 128 * _NT + vt % 40
    a_tail = vt // 40
    tail = xi >= 512 * _NT
    r = jnp.where(tail, r_tail, r_main)
    a = jnp.where(tail, a_tail, a_main)
    tabrow = jnp.arange(N_FIELDS, dtype=jnp.int32) * VOCAB + r * 4 + a
    idx = (
        tabrow.reshape(_NP, 2 * N_FIELDS)[:, jnp.array(_PERM52)]
        .reshape(_NP, _NJ, 4)
        .transpose(1, 0, 2)
        .reshape(_NW, _NG, _G)
    )
    emb = _sc_gather(tab_flat, idx)
    emb2 = emb.reshape(_NJ * _NP, 128)
    W2 = jnp.zeros((128 * _NJ, 2 * OUT_FEATURES), jnp.float32)
    W2 = W2.at[:IN_FEAT, :OUT_FEATURES].set(W)
    W2 = W2.at[IN_FEAT:, OUT_FEATURES:].set(W)
    b2 = jnp.concatenate([b, b]).reshape(1, 2 * OUT_FEATURES)
    return _tc_matmul(emb2, W2, b2)


# speedup vs baseline: 1.0942x; 1.0942x over previous
import functools

import jax
import jax.numpy as jnp
from jax import lax
from jax.experimental import pallas as pl
from jax.experimental.pallas import tpu as pltpu
from jax.experimental.pallas import tpu_sc as plsc

N_FIELDS = 26
VOCAB = 100000
EMB_DIM = 32
BATCH = 16384
OUT_FEATURES = 128
IN_FEAT = N_FIELDS * EMB_DIM

_FH = N_FIELDS // 2
_NW = 32
_ROWS = BATCH * _FH
_RPW = _ROWS // _NW
_G = 128
_NG = _RPW // _G
_NPAIR = _NG // 2


def _gather_body(tab_hbm, idx_hbm, out_hbm, idx_v, buf0, buf1, sem0, sem1):
    nc = lax.axis_size("c")
    wid = lax.axis_index("s") * nc + lax.axis_index("c")
    pltpu.sync_copy(idx_hbm.at[wid], idx_v)
    base = wid * _RPW

    pltpu.async_copy(tab_hbm.at[idx_v.at[0]], buf0, sem0)

    def body(i, carry):
        a = 2 * i
        pltpu.async_copy(tab_hbm.at[idx_v.at[a + 1]], buf1, sem1)
        pltpu.make_async_copy(tab_hbm.at[idx_v.at[a]], buf0, sem0).wait()
        pltpu.sync_copy(buf0, out_hbm.at[pl.ds(base + a * _G, _G)])

        @pl.when(i < _NPAIR - 1)
        def _():
            pltpu.async_copy(tab_hbm.at[idx_v.at[a + 2]], buf0, sem0)

        pltpu.make_async_copy(tab_hbm.at[idx_v.at[a + 1]], buf1, sem1).wait()
        pltpu.sync_copy(buf1, out_hbm.at[pl.ds(base + (a + 1) * _G, _G)])
        return carry

    lax.fori_loop(0, _NPAIR, body, 0)


@functools.partial(
    pl.kernel,
    out_type=jax.ShapeDtypeStruct((_ROWS, EMB_DIM), jnp.float32),
    mesh=plsc.VectorSubcoreMesh(core_axis_name="c", subcore_axis_name="s"),
    scratch_types=[
        pltpu.VMEM((_NG, _G), jnp.int32),
        pltpu.VMEM((_G, EMB_DIM), jnp.float32),
        pltpu.VMEM((_G, EMB_DIM), jnp.float32),
        pltpu.SemaphoreType.DMA,
        pltpu.SemaphoreType.DMA,
    ],
    compiler_params=pltpu.CompilerParams(use_tc_tiling_on_sc=False),
)
def _sc_gather(tab_hbm, idx_hbm, out_hbm, idx_v, buf0, buf1, sem0, sem1):
    _gather_body(tab_hbm, idx_hbm, out_hbm, idx_v, buf0, buf1, sem0, sem1)


_VQ = VOCAB // 4


_NT = VOCAB // 512


def _tr_body(in_ref, out_ref):
    ident = jnp.eye(128, dtype=jnp.float32)
    dn = (((0,), (0,)), ((), ()))

    def body(i, carry):
        for u in range(14):
            t = 14 * i + u
            base = 512 * t
            xs = jnp.concatenate(
                [in_ref[0, :, pl.ds(base + 128 * a, 128)] for a in range(4)],
                axis=0,
            )
            out_ref[pl.ds(128 * t, 128), :] = lax.dot_general(
                xs, ident, dn, preferred_element_type=jnp.float32
            )
        return carry

    lax.fori_loop(0, _NT // 14, body, 0)
    for t in range(14 * (_NT // 14), _NT):
        base = 512 * t
        xs = jnp.concatenate(
            [in_ref[0, :, base + 128 * a:base + 128 * (a + 1)]
             for a in range(4)],
            axis=0,
        )
        out_ref[128 * t:128 * (t + 1), :] = lax.dot_general(
            xs, ident, dn, preferred_element_type=jnp.float32
        )
    tb = 512 * _NT
    xt = jnp.concatenate(
        [in_ref[0, :, tb + 40 * a:tb + 40 * (a + 1)] for a in range(4)],
        axis=0,
    )
    out_ref[128 * _NT:_VQ, :] = lax.dot_general(
        xt, ident, dn, preferred_element_type=jnp.float32
    )


def _tc_transpose(tabT, half):
    rows_per_field = _VQ
    return pl.pallas_call(
        _tr_body,
        grid=(_FH,),
        in_specs=[pl.BlockSpec((1, EMB_DIM, VOCAB),
                               lambda f: (f + half * _FH, 0, 0))],
        out_specs=pl.BlockSpec((rows_per_field, 128), lambda f: (f, 0)),
        out_shape=jax.ShapeDtypeStruct((_FH * rows_per_field, 128),
                                       jnp.float32),
    )(tabT)


def _mm_body(ea_ref, eb_ref, wa_ref, wb_ref, b_ref, o_ref):
    o_ref[...] = (
        jnp.dot(ea_ref[...], wa_ref[...], preferred_element_type=jnp.float32)
        + jnp.dot(eb_ref[...], wb_ref[...], preferred_element_type=jnp.float32)
        + b_ref[...]
    )


_BM = 2048
_HF = _FH * EMB_DIM


def _tc_matmul(embA, embB, W, b):
    return pl.pallas_call(
        _mm_body,
        grid=(BATCH // _BM,),
        in_specs=[
            pl.BlockSpec((_BM, _HF), lambda i: (i, 0)),
            pl.BlockSpec((_BM, _HF), lambda i: (i, 0)),
            pl.BlockSpec((_HF, OUT_FEATURES), lambda i: (0, 0)),
            pl.BlockSpec((_HF, OUT_FEATURES), lambda i: (0, 0)),
            pl.BlockSpec((1, OUT_FEATURES), lambda i: (0, 0)),
        ],
        out_specs=pl.BlockSpec((_BM, OUT_FEATURES), lambda i: (i, 0)),
        out_shape=jax.ShapeDtypeStruct((BATCH, OUT_FEATURES), jnp.float32),
    )(embA, embB, W[:_HF], W[_HF:], b.reshape(1, OUT_FEATURES))


def kernel(x, tables, W, b):
    tabT = jnp.transpose(tables, (0, 2, 1))
    xi = x.astype(jnp.int32)
    vt = xi - 512 * _NT
    r_main = 128 * (xi // 512) + xi % 128
    a_main = (xi // 128) % 4
    r_tail = 128 * _NT + vt % 40
    a_tail = vt // 40
    tail = xi >= 512 * _NT
    r = jnp.where(tail, r_tail, r_main)
    a = jnp.where(tail, a_tail, a_main)
    rowp = r * 4 + a
    foff = jnp.arange(_FH, dtype=jnp.int32) * VOCAB
    embs = []
    for half in range(2):
        tab_flat = _tc_transpose(tabT, half).reshape(_FH * VOCAB, EMB_DIM)
        idx = foff + rowp[:, half * _FH:(half + 1) * _FH]
        emb = _sc_gather(tab_flat, idx.reshape(_NW, _NG, _G))
        embs.append(emb.reshape(BATCH, _HF))
    return _tc_matmul(embs[0], embs[1], W, b)


# speedup vs baseline: 1.1131x; 1.0173x over previous
import functools

import jax
import jax.numpy as jnp
from jax import lax
from jax.experimental import pallas as pl
from jax.experimental.pallas import tpu as pltpu
from jax.experimental.pallas import tpu_sc as plsc

N_FIELDS = 26
VOCAB = 100000
EMB_DIM = 32
BATCH = 16384
OUT_FEATURES = 128
IN_FEAT = N_FIELDS * EMB_DIM

_FA = 18
_FB = N_FIELDS - _FA
_NW = 32
_G = 128


def _make_sc_gather(nf):
    rows = BATCH * nf
    rpw = rows // _NW
    ng = rpw // _G
    npair = ng // 2

    def body_fn(tab_hbm, idx_hbm, out_hbm, idx_v, buf0, buf1, sem0, sem1):
        nc = lax.axis_size("c")
        wid = lax.axis_index("s") * nc + lax.axis_index("c")
        pltpu.sync_copy(idx_hbm.at[wid], idx_v)
        base = wid * rpw

        pltpu.async_copy(tab_hbm.at[idx_v.at[0]], buf0, sem0)

        def body(i, carry):
            a = 2 * i
            pltpu.async_copy(tab_hbm.at[idx_v.at[a + 1]], buf1, sem1)
            pltpu.make_async_copy(tab_hbm.at[idx_v.at[a]], buf0, sem0).wait()
            pltpu.sync_copy(buf0, out_hbm.at[pl.ds(base + a * _G, _G)])

            @pl.when(i < npair - 1)
            def _():
                pltpu.async_copy(tab_hbm.at[idx_v.at[a + 2]], buf0, sem0)

            pltpu.make_async_copy(tab_hbm.at[idx_v.at[a + 1]], buf1,
                                  sem1).wait()
            pltpu.sync_copy(buf1, out_hbm.at[pl.ds(base + (a + 1) * _G, _G)])
            return carry

        lax.fori_loop(0, npair, body, 0)

    return pl.kernel(
        body_fn,
        out_type=jax.ShapeDtypeStruct((rows, EMB_DIM), jnp.float32),
        mesh=plsc.VectorSubcoreMesh(core_axis_name="c", subcore_axis_name="s"),
        scratch_types=[
            pltpu.VMEM((ng, _G), jnp.int32),
            pltpu.VMEM((_G, EMB_DIM), jnp.float32),
            pltpu.VMEM((_G, EMB_DIM), jnp.float32),
            pltpu.SemaphoreType.DMA,
            pltpu.SemaphoreType.DMA,
        ],
        compiler_params=pltpu.CompilerParams(use_tc_tiling_on_sc=False),
    )


_sc_gather_a = _make_sc_gather(_FA)
_sc_gather_b = _make_sc_gather(_FB)


_VQ = VOCAB // 4


_NT = VOCAB // 512


def _tr_body(in_ref, out_ref):
    ident = jnp.eye(128, dtype=jnp.float32)
    dn = (((0,), (0,)), ((), ()))

    def body(i, carry):
        for u in range(14):
            t = 14 * i + u
            base = 512 * t
            xs = jnp.concatenate(
                [in_ref[0, :, pl.ds(base + 128 * a, 128)] for a in range(4)],
                axis=0,
            )
            out_ref[pl.ds(128 * t, 128), :] = lax.dot_general(
                xs, ident, dn, preferred_element_type=jnp.float32
            )
        return carry

    lax.fori_loop(0, _NT // 14, body, 0)
    for t in range(14 * (_NT // 14), _NT):
        base = 512 * t
        xs = jnp.concatenate(
            [in_ref[0, :, base + 128 * a:base + 128 * (a + 1)]
             for a in range(4)],
            axis=0,
        )
        out_ref[128 * t:128 * (t + 1), :] = lax.dot_general(
            xs, ident, dn, preferred_element_type=jnp.float32
        )
    tb = 512 * _NT
    xt = jnp.concatenate(
        [in_ref[0, :, tb + 40 * a:tb + 40 * (a + 1)] for a in range(4)],
        axis=0,
    )
    out_ref[128 * _NT:_VQ, :] = lax.dot_general(
        xt, ident, dn, preferred_element_type=jnp.float32
    )


def _tc_transpose(tabT, off, nf):
    rows_per_field = _VQ
    return pl.pallas_call(
        _tr_body,
        grid=(nf,),
        in_specs=[pl.BlockSpec((1, EMB_DIM, VOCAB),
                               lambda f: (f + off, 0, 0))],
        out_specs=pl.BlockSpec((rows_per_field, 128), lambda f: (f, 0)),
        out_shape=jax.ShapeDtypeStruct((nf * rows_per_field, 128),
                                       jnp.float32),
    )(tabT)


def _mm_body(ea_ref, eb_ref, wa_ref, wb_ref, b_ref, o_ref):
    o_ref[...] = (
        jnp.dot(ea_ref[...], wa_ref[...], preferred_element_type=jnp.float32)
        + jnp.dot(eb_ref[...], wb_ref[...], preferred_element_type=jnp.float32)
        + b_ref[...]
    )


_BM = 2048
_HA = _FA * EMB_DIM
_HB = _FB * EMB_DIM


def _tc_matmul(embA, embB, W, b):
    return pl.pallas_call(
        _mm_body,
        grid=(BATCH // _BM,),
        in_specs=[
            pl.BlockSpec((_BM, _HA), lambda i: (i, 0)),
            pl.BlockSpec((_BM, _HB), lambda i: (i, 0)),
            pl.BlockSpec((_HA, OUT_FEATURES), lambda i: (0, 0)),
            pl.BlockSpec((_HB, OUT_FEATURES), lambda i: (0, 0)),
            pl.BlockSpec((1, OUT_FEATURES), lambda i: (0, 0)),
        ],
        out_specs=pl.BlockSpec((_BM, OUT_FEATURES), lambda i: (i, 0)),
        out_shape=jax.ShapeDtypeStruct((BATCH, OUT_FEATURES), jnp.float32),
    )(embA, embB, W[:_HA], W[_HA:], b.reshape(1, OUT_FEATURES))


def kernel(x, tables, W, b):
    tabT = jnp.transpose(tables, (0, 2, 1))
    xi = x.astype(jnp.int32)
    vt = xi - 512 * _NT
    r_main = 128 * (xi // 512) + xi % 128
    a_main = (xi // 128) % 4
    r_tail = 128 * _NT + vt % 40
    a_tail = vt // 40
    tail = xi >= 512 * _NT
    r = jnp.where(tail, r_tail, r_main)
    a = jnp.where(tail, a_tail, a_main)
    rowp = r * 4 + a
    embs = []
    for off, nf, gather in ((0, _FA, _sc_gather_a), (_FA, _FB, _sc_gather_b)):
        tab_flat = _tc_transpose(tabT, off, nf).reshape(nf * VOCAB, EMB_DIM)
        idx = (jnp.arange(nf, dtype=jnp.int32) * VOCAB
               + rowp[:, off:off + nf])
        emb = gather(tab_flat, idx.reshape(_NW, 4 * nf, _G))
        embs.append(emb.reshape(BATCH, nf * EMB_DIM))
    return _tc_matmul(embs[0], embs[1], W, b)


# speedup vs baseline: 1.1142x; 1.0009x over previous
import jax
import jax.numpy as jnp
from jax import lax
from jax.experimental import pallas as pl
from jax.experimental.pallas import tpu as pltpu
from jax.experimental.pallas import tpu_sc as plsc

N_FIELDS = 26
VOCAB = 100000
EMB_DIM = 32
BATCH = 16384
OUT_FEATURES = 128
IN_FEAT = N_FIELDS * EMB_DIM

_FA = 18
_FB = N_FIELDS - _FA
_NW = 32
_G = 128


def _make_sc_gather(nf):
    rows = BATCH * nf
    rpw = rows // _NW
    ng = rpw // _G
    npair = ng // 2

    def body_fn(tab_hbm, idx_hbm, out_hbm, idx_v, buf0, buf1, sem0, sem1):
        nc = lax.axis_size("c")
        wid = lax.axis_index("s") * nc + lax.axis_index("c")
        pltpu.sync_copy(idx_hbm.at[wid], idx_v)
        base = wid * rpw

        pltpu.async_copy(tab_hbm.at[idx_v.at[0]], buf0, sem0)

        def body(i, carry):
            a = 2 * i
            pltpu.async_copy(tab_hbm.at[idx_v.at[a + 1]], buf1, sem1)
            pltpu.make_async_copy(tab_hbm.at[idx_v.at[a]], buf0, sem0).wait()
            pltpu.sync_copy(buf0, out_hbm.at[pl.ds(base + a * _G, _G)])

            @pl.when(i < npair - 1)
            def _():
                pltpu.async_copy(tab_hbm.at[idx_v.at[a + 2]], buf0, sem0)

            pltpu.make_async_copy(tab_hbm.at[idx_v.at[a + 1]], buf1,
                                  sem1).wait()
            pltpu.sync_copy(buf1, out_hbm.at[pl.ds(base + (a + 1) * _G, _G)])
            return carry

        lax.fori_loop(0, npair, body, 0)

    return pl.kernel(
        body_fn,
        out_type=jax.ShapeDtypeStruct((rows, EMB_DIM), jnp.float32),
        mesh=plsc.VectorSubcoreMesh(core_axis_name="c", subcore_axis_name="s"),
        scratch_types=[
            pltpu.VMEM((ng, _G), jnp.int32),
            pltpu.VMEM((_G, EMB_DIM), jnp.float32),
            pltpu.VMEM((_G, EMB_DIM), jnp.float32),
            pltpu.SemaphoreType.DMA,
            pltpu.SemaphoreType.DMA,
        ],
        compiler_params=pltpu.CompilerParams(use_tc_tiling_on_sc=False),
    )


_sc_gather_a = _make_sc_gather(_FA)
_sc_gather_b = _make_sc_gather(_FB)


_VQ = VOCAB // 4


_NT = VOCAB // 512


def _tr_body(in_ref, out_ref):
    ident = jnp.eye(128, dtype=jnp.float32)
    dn = (((0,), (0,)), ((), ()))

    def body(i, carry):
        for u in range(14):
            t = 14 * i + u
            base = 512 * t
            xs = jnp.concatenate(
                [in_ref[0, :, pl.ds(base + 128 * a, 128)] for a in range(4)],
                axis=0,
            )
            out_ref[pl.ds(128 * t, 128), :] = lax.dot_general(
                xs, ident, dn, preferred_element_type=jnp.float32
            )
        return carry

    lax.fori_loop(0, _NT // 14, body, 0)
    for t in range(14 * (_NT // 14), _NT):
        base = 512 * t
        xs = jnp.concatenate(
            [in_ref[0, :, base + 128 * a:base + 128 * (a + 1)]
             for a in range(4)],
            axis=0,
        )
        out_ref[128 * t:128 * (t + 1), :] = lax.dot_general(
            xs, ident, dn, preferred_element_type=jnp.float32
        )
    tb = 512 * _NT
    xt = jnp.concatenate(
        [in_ref[0, :, tb + 40 * a:tb + 40 * (a + 1)] for a in range(4)],
        axis=0,
    )
    out_ref[128 * _NT:_VQ, :] = lax.dot_general(
        xt, ident, dn, preferred_element_type=jnp.float32
    )


def _tc_transpose(tabT, off, nf):
    rows_per_field = _VQ
    return pl.pallas_call(
        _tr_body,
        grid=(nf,),
        in_specs=[pl.BlockSpec((1, EMB_DIM, VOCAB),
                               lambda f: (f + off, 0, 0))],
        out_specs=pl.BlockSpec((rows_per_field, 128), lambda f: (f, 0)),
        out_shape=jax.ShapeDtypeStruct((nf * rows_per_field, 128),
                                       jnp.float32),
    )(tabT)


def _mm_body(ea_ref, eb_ref, wa_ref, wb_ref, b_ref, o_ref):
    o_ref[...] = (
        jnp.dot(ea_ref[...], wa_ref[...], preferred_element_type=jnp.float32)
        + jnp.dot(eb_ref[...], wb_ref[...], preferred_element_type=jnp.float32)
        + b_ref[...]
    )


_BM = 2048
_HA = _FA * EMB_DIM
_HB = _FB * EMB_DIM


def _tc_matmul(embA, embB, W, b):
    return pl.pallas_call(
        _mm_body,
        grid=(BATCH // _BM,),
        in_specs=[
            pl.BlockSpec((_BM, _HA), lambda i: (i, 0)),
            pl.BlockSpec((_BM, _HB), lambda i: (i, 0)),
            pl.BlockSpec((_HA, OUT_FEATURES), lambda i: (0, 0)),
            pl.BlockSpec((_HB, OUT_FEATURES), lambda i: (0, 0)),
            pl.BlockSpec((1, OUT_FEATURES), lambda i: (0, 0)),
        ],
        out_specs=pl.BlockSpec((_BM, OUT_FEATURES), lambda i: (i, 0)),
        out_shape=jax.ShapeDtypeStruct((BATCH, OUT_FEATURES), jnp.float32),
    )(embA, embB, W[:_HA], W[_HA:], b.reshape(1, OUT_FEATURES))


def kernel(x, tables, W, b):
    tabT = jnp.transpose(tables, (0, 2, 1))
    xi = x.astype(jnp.int32)
    vt = xi - 512 * _NT
    r_main = 128 * (xi // 512) + xi % 128
    a_main = (xi // 128) % 4
    r_tail = 128 * _NT + vt % 40
    a_tail = vt // 40
    tail = xi >= 512 * _NT
    r = jnp.where(tail, r_tail, r_main)
    a = jnp.where(tail, a_tail, a_main)
    rowp = r * 4 + a
    embs = []
    for off, nf, gather in ((0, _FA, _sc_gather_a), (_FA, _FB, _sc_gather_b)):
        tab_flat = _tc_transpose(tabT, off, nf).reshape(nf * VOCAB, EMB_DIM)
        idx = (jnp.arange(nf, dtype=jnp.int32) * VOCAB
               + rowp[:, off:off + nf])
        emb = gather(tab_flat, idx.reshape(_NW, 4 * nf, _G))
        embs.append(emb.reshape(BATCH, nf * EMB_DIM))
    return _tc_matmul(embs[0], embs[1], W, b)


# speedup vs baseline: 1.1164x; 1.0020x over previous
import jax
import jax.numpy as jnp
from jax import lax
from jax.experimental import pallas as pl
from jax.experimental.pallas import tpu as pltpu
from jax.experimental.pallas import tpu_sc as plsc

N_FIELDS = 26
VOCAB = 100000
EMB_DIM = 32
BATCH = 16384
OUT_FEATURES = 128
IN_FEAT = N_FIELDS * EMB_DIM

_FA = 20
_FB = N_FIELDS - _FA
_NW = 32
_G = 128


def _make_sc_gather(nf):
    rows = BATCH * nf
    rpw = rows // _NW
    ng = rpw // _G
    npair = ng // 2

    def body_fn(tab_hbm, idx_hbm, out_hbm, idx_v, buf0, buf1, sem0, sem1):
        nc = lax.axis_size("c")
        wid = lax.axis_index("s") * nc + lax.axis_index("c")
        pltpu.sync_copy(idx_hbm.at[wid], idx_v)
        base = wid * rpw

        pltpu.async_copy(tab_hbm.at[idx_v.at[0]], buf0, sem0)

        def body(i, carry):
            a = 2 * i
            pltpu.async_copy(tab_hbm.at[idx_v.at[a + 1]], buf1, sem1)
            pltpu.make_async_copy(tab_hbm.at[idx_v.at[a]], buf0, sem0).wait()
            pltpu.sync_copy(buf0, out_hbm.at[pl.ds(base + a * _G, _G)])

            @pl.when(i < npair - 1)
            def _():
                pltpu.async_copy(tab_hbm.at[idx_v.at[a + 2]], buf0, sem0)

            pltpu.make_async_copy(tab_hbm.at[idx_v.at[a + 1]], buf1,
                                  sem1).wait()
            pltpu.sync_copy(buf1, out_hbm.at[pl.ds(base + (a + 1) * _G, _G)])
            return carry

        lax.fori_loop(0, npair, body, 0)

    return pl.kernel(
        body_fn,
        out_type=jax.ShapeDtypeStruct((rows, EMB_DIM), jnp.float32),
        mesh=plsc.VectorSubcoreMesh(core_axis_name="c", subcore_axis_name="s"),
        scratch_types=[
            pltpu.VMEM((ng, _G), jnp.int32),
            pltpu.VMEM((_G, EMB_DIM), jnp.float32),
            pltpu.VMEM((_G, EMB_DIM), jnp.float32),
            pltpu.SemaphoreType.DMA,
            pltpu.SemaphoreType.DMA,
        ],
        compiler_params=pltpu.CompilerParams(use_tc_tiling_on_sc=False),
    )


_sc_gather_a = _make_sc_gather(_FA)
_sc_gather_b = _make_sc_gather(_FB)


_VQ = VOCAB // 4


_NT = VOCAB // 512


def _tr_body(in_ref, out_ref):
    ident = jnp.eye(128, dtype=jnp.float32)
    dn = (((0,), (0,)), ((), ()))

    def body(i, carry):
        for u in range(14):
            t = 14 * i + u
            base = 512 * t
            xs = jnp.concatenate(
                [in_ref[0, :, pl.ds(base + 128 * a, 128)] for a in range(4)],
                axis=0,
            )
            out_ref[pl.ds(128 * t, 128), :] = lax.dot_general(
                xs, ident, dn, preferred_element_type=jnp.float32
            )
        return carry

    lax.fori_loop(0, _NT // 14, body, 0)
    for t in range(14 * (_NT // 14), _NT):
        base = 512 * t
        xs = jnp.concatenate(
            [in_ref[0, :, base + 128 * a:base + 128 * (a + 1)]
             for a in range(4)],
            axis=0,
        )
        out_ref[128 * t:128 * (t + 1), :] = lax.dot_general(
            xs, ident, dn, preferred_element_type=jnp.float32
        )
    tb = 512 * _NT
    xt = jnp.concatenate(
        [in_ref[0, :, tb + 40 * a:tb + 40 * (a + 1)] for a in range(4)],
        axis=0,
    )
    out_ref[128 * _NT:_VQ, :] = lax.dot_general(
        xt, ident, dn, preferred_element_type=jnp.float32
    )


def _tc_transpose(tabT, off, nf):
    rows_per_field = _VQ
    return pl.pallas_call(
        _tr_body,
        grid=(nf,),
        in_specs=[pl.BlockSpec((1, EMB_DIM, VOCAB),
                               lambda f: (f + off, 0, 0))],
        out_specs=pl.BlockSpec((rows_per_field, 128), lambda f: (f, 0)),
        out_shape=jax.ShapeDtypeStruct((nf * rows_per_field, 128),
                                       jnp.float32),
    )(tabT)


def _mm_body(ea_ref, eb_ref, wa_ref, wb_ref, b_ref, o_ref):
    o_ref[...] = (
        jnp.dot(ea_ref[...], wa_ref[...], preferred_element_type=jnp.float32)
        + jnp.dot(eb_ref[...], wb_ref[...], preferred_element_type=jnp.float32)
        + b_ref[...]
    )


_BM = 2048
_HA = _FA * EMB_DIM
_HB = _FB * EMB_DIM


def _tc_matmul(embA, embB, W, b):
    return pl.pallas_call(
        _mm_body,
        grid=(BATCH // _BM,),
        in_specs=[
            pl.BlockSpec((_BM, _HA), lambda i: (i, 0)),
            pl.BlockSpec((_BM, _HB), lambda i: (i, 0)),
            pl.BlockSpec((_HA, OUT_FEATURES), lambda i: (0, 0)),
            pl.BlockSpec((_HB, OUT_FEATURES), lambda i: (0, 0)),
            pl.BlockSpec((1, OUT_FEATURES), lambda i: (0, 0)),
        ],
        out_specs=pl.BlockSpec((_BM, OUT_FEATURES), lambda i: (i, 0)),
        out_shape=jax.ShapeDtypeStruct((BATCH, OUT_FEATURES), jnp.float32),
    )(embA, embB, W[:_HA], W[_HA:], b.reshape(1, OUT_FEATURES))


def kernel(x, tables, W, b):
    tabT = jnp.transpose(tables, (0, 2, 1))
    xi = x.astype(jnp.int32)
    vt = xi - 512 * _NT
    r_main = 128 * (xi // 512) + xi % 128
    a_main = (xi // 128) % 4
    r_tail = 128 * _NT + vt % 40
    a_tail = vt // 40
    tail = xi >= 512 * _NT
    r = jnp.where(tail, r_tail, r_main)
    a = jnp.where(tail, a_tail, a_main)
    rowp = r * 4 + a
    embs = []
    for off, nf, gather in ((0, _FA, _sc_gather_a), (_FA, _FB, _sc_gather_b)):
        tab_flat = _tc_transpose(tabT, off, nf).reshape(nf * VOCAB, EMB_DIM)
        idx = (jnp.arange(nf, dtype=jnp.int32) * VOCAB
               + rowp[:, off:off + nf])
        emb = gather(tab_flat, idx.reshape(_NW, 4 * nf, _G))
        embs.append(emb.reshape(BATCH, nf * EMB_DIM))
    return _tc_matmul(embs[0], embs[1], W, b)
